# Initial kernel scaffold; baseline (speedup 1.0000x reference)
import jax
import jax.numpy as jnp
from jax.experimental import pallas as pl


def kernel(weights, existing_bins, nears, fars):
    raise NotImplementedError("write your pallas kernel here")


# speedup vs baseline: 4.5927x; 4.5927x over previous
import functools

import jax
import jax.numpy as jnp
from jax import lax
from jax.experimental import pallas as pl
from jax.experimental.pallas import tpu as pltpu
from jax.experimental.pallas import tpu_sc as plsc

NUM_RAYS = 32768
NS = 64
NB = NS + 1
NOUT = 2 * NB
L = 16

_info = plsc.get_sparse_core_info()
NWORK = _info.num_cores * _info.num_subcores
RAYS_PER_W = NUM_RAYS // NWORK
GROUPS = RAYS_PER_W // L


def _body(w_hbm, e_hbm, near_hbm, far_hbm, out_hbm,
          w_v, e_v, near_v, far_v, cdf_t, hist, out_v):
    nc = _info.num_cores
    wid = lax.axis_index("s") * nc + lax.axis_index("c")
    lanes = lax.iota(jnp.int32, L)
    onesf = jnp.full((L,), 1.0, jnp.float32)
    zerosf = jnp.zeros((L,), jnp.float32)
    cdf_t[0] = zerosf

    def group(g, carry):
        base = (wid * GROUPS + g) * L
        pltpu.sync_copy(w_hbm.at[pl.ds(base, L), :], w_v)
        pltpu.sync_copy(e_hbm.at[pl.ds(base, L), :], e_v)
        pltpu.sync_copy(near_hbm.at[pl.ds(base, L)], near_v)
        pltpu.sync_copy(far_hbm.at[pl.ds(base, L)], far_v)

        near = near_v[...]
        fn = far_v[...] - near

        c = zerosf
        for i in range(NS):
            wi = plsc.load_gather(w_v, [lanes, jnp.full((L,), i, jnp.int32)])
            c = c + (wi + 0.01)
            cdf_t[i + 1] = c
        ws = c
        pad = jnp.maximum(1e-5 - ws, 0.0)
        off = pad * (1.0 / NS)
        r = 1.0 / (ws + pad)

        hist[0] = onesf
        for i in range(1, NB + 1):
            hist[i] = zerosf

        e0 = plsc.load_gather(e_v, [lanes, jnp.zeros((L,), jnp.int32)])
        plsc.store_scatter(out_v, [lanes, jnp.zeros((L,), jnp.int32)],
                           e0 * fn + near)

        for i in range(1, NB):
            cdf = jnp.minimum(1.0, (cdf_t[i] + off * float(i)) * r)
            cdf_t[i] = cdf
            x = cdf * float(NB) - 0.5
            ti = x.astype(jnp.int32)
            p = ti + (x > ti.astype(jnp.float32)).astype(jnp.int32)
            plsc.addupdate_scatter(hist, [p, lanes], onesf)
            ei = plsc.load_gather(e_v, [lanes, jnp.full((L,), i, jnp.int32)])
            plsc.store_scatter(out_v, [lanes, p + i], ei * fn + near)

        run = zerosf
        for j in range(NB):
            run = run + hist[j]
            below = run.astype(jnp.int32) - 1
            above = jnp.minimum(below + 1, NS)
            g0 = plsc.load_gather(cdf_t, [below, lanes])
            g1 = plsc.load_gather(cdf_t, [above, lanes])
            b0 = plsc.load_gather(e_v, [lanes, below])
            b1 = plsc.load_gather(e_v, [lanes, above])
            denom = g1 - g0
            ok = denom > 1e-12
            sd = jnp.where(ok, denom, 1.0)
            t = jnp.where(ok, ((j + 0.5) * (1.0 / NB) - g0) / sd, 0.0)
            t = jnp.clip(t, 0.0, 1.0)
            bins = b0 + t * (b1 - b0)
            plsc.store_scatter(out_v, [lanes, below + (j + 1)],
                               bins * fn + near)

        pltpu.sync_copy(out_v, out_hbm.at[pl.ds(base, L), :])
        return carry

    lax.fori_loop(0, GROUPS, group, 0)


@jax.jit
def _run(w2, e2, n1, f1):
    mesh = plsc.VectorSubcoreMesh(core_axis_name="c", subcore_axis_name="s")
    fn = pl.kernel(
        _body,
        out_type=jax.ShapeDtypeStruct((NUM_RAYS, NOUT), jnp.float32),
        mesh=mesh,
        compiler_params=pltpu.CompilerParams(needs_layout_passes=False),
        scratch_types=[
            pltpu.VMEM((L, NS), jnp.float32),
            pltpu.VMEM((L, NB), jnp.float32),
            pltpu.VMEM((L,), jnp.float32),
            pltpu.VMEM((L,), jnp.float32),
            pltpu.VMEM((NB, L), jnp.float32),
            pltpu.VMEM((NB + 1, L), jnp.float32),
            pltpu.VMEM((L, NOUT), jnp.float32),
        ],
    )
    return fn(w2, e2, n1, f1)


def kernel(weights, existing_bins, nears, fars):
    return _run(weights[..., 0], existing_bins, nears[:, 0], fars[:, 0])


# speedup vs baseline: 5.3464x; 1.1641x over previous
import functools

import jax
import jax.numpy as jnp
from jax import lax
from jax.experimental import pallas as pl
from jax.experimental.pallas import tpu as pltpu
from jax.experimental.pallas import tpu_sc as plsc

NUM_RAYS = 32768
NS = 64
NB = NS + 1
NOUT = 2 * NB
L = 16

_info = plsc.get_sparse_core_info()
NWORK = _info.num_cores * _info.num_subcores
RAYS_PER_W = NUM_RAYS // NWORK

CHUNK = 128
GPC = CHUNK // L
NCHUNK = RAYS_PER_W // CHUNK


def _body(w_hbm, e_hbm, near_hbm, far_hbm, out_hbm,
          w_c, e_c, near_c, far_c, out_c, cdf_t, hist):
    nc = _info.num_cores
    wid = lax.axis_index("s") * nc + lax.axis_index("c")
    lanes = lax.iota(jnp.int32, L)
    onesf = jnp.full((L,), 1.0, jnp.float32)
    zerosf = jnp.zeros((L,), jnp.float32)
    cdf_t[0] = zerosf

    def chunk(cidx, carry):
        base = wid * RAYS_PER_W + cidx * CHUNK
        pltpu.sync_copy(w_hbm.at[pl.ds(base, CHUNK), :], w_c)
        pltpu.sync_copy(e_hbm.at[pl.ds(base, CHUNK), :], e_c)
        pltpu.sync_copy(near_hbm.at[pl.ds(base, CHUNK)], near_c)
        pltpu.sync_copy(far_hbm.at[pl.ds(base, CHUNK)], far_c)

        def group(g, carry2):
            off = g * L
            rows = off + lanes
            near = near_c[pl.ds(off, L)]
            fn = far_c[pl.ds(off, L)] - near

            c = zerosf
            for i in range(NS):
                wi = plsc.load_gather(w_c, [rows, jnp.full((L,), i, jnp.int32)])
                c = c + (wi + 0.01)
                cdf_t[i + 1] = c
            ws = c
            pad = jnp.maximum(1e-5 - ws, 0.0)
            off_w = pad * (1.0 / NS)
            r = 1.0 / (ws + pad)

            hist[0] = onesf
            for i in range(1, NB + 1):
                hist[i] = zerosf

            e0 = plsc.load_gather(e_c, [rows, jnp.zeros((L,), jnp.int32)])
            plsc.store_scatter(out_c, [rows, jnp.zeros((L,), jnp.int32)],
                               e0 * fn + near)

            for i in range(1, NB):
                cdf = jnp.minimum(1.0, (cdf_t[i] + off_w * float(i)) * r)
                cdf_t[i] = cdf
                x = cdf * float(NB) - 0.5
                ti = x.astype(jnp.int32)
                p = ti + (x > ti.astype(jnp.float32)).astype(jnp.int32)
                plsc.addupdate_scatter(hist, [p, lanes], onesf)
                ei = plsc.load_gather(e_c, [rows, jnp.full((L,), i, jnp.int32)])
                plsc.store_scatter(out_c, [rows, p + i], ei * fn + near)

            run = zerosf
            for j in range(NB):
                run = run + hist[j]
                below = run.astype(jnp.int32) - 1
                above = jnp.minimum(below + 1, NS)
                g0 = plsc.load_gather(cdf_t, [below, lanes])
                g1 = plsc.load_gather(cdf_t, [above, lanes])
                b0 = plsc.load_gather(e_c, [rows, below])
                b1 = plsc.load_gather(e_c, [rows, above])
                denom = g1 - g0
                ok = denom > 1e-12
                sd = jnp.where(ok, denom, 1.0)
                t = jnp.where(ok, ((j + 0.5) * (1.0 / NB) - g0) / sd, 0.0)
                t = jnp.clip(t, 0.0, 1.0)
                bins = b0 + t * (b1 - b0)
                plsc.store_scatter(out_c, [rows, below + (j + 1)],
                                   bins * fn + near)
            return carry2

        lax.fori_loop(0, GPC, group, 0)
        pltpu.sync_copy(out_c, out_hbm.at[pl.ds(base, CHUNK), :])
        return carry

    lax.fori_loop(0, NCHUNK, chunk, 0)


@jax.jit
def _run(w2, e2, n1, f1):
    mesh = plsc.VectorSubcoreMesh(core_axis_name="c", subcore_axis_name="s")
    fn = pl.kernel(
        _body,
        out_type=jax.ShapeDtypeStruct((NUM_RAYS, NOUT), jnp.float32),
        mesh=mesh,
        compiler_params=pltpu.CompilerParams(needs_layout_passes=False),
        scratch_types=[
            pltpu.VMEM((CHUNK, NS), jnp.float32),
            pltpu.VMEM((CHUNK, NB), jnp.float32),
            pltpu.VMEM((CHUNK,), jnp.float32),
            pltpu.VMEM((CHUNK,), jnp.float32),
            pltpu.VMEM((CHUNK, NOUT), jnp.float32),
            pltpu.VMEM((NB, L), jnp.float32),
            pltpu.VMEM((NB + 1, L), jnp.float32),
        ],
    )
    return fn(w2, e2, n1, f1)


def kernel(weights, existing_bins, nears, fars):
    return _run(weights[..., 0], existing_bins, nears[:, 0], fars[:, 0])
